# Initial kernel scaffold; baseline (speedup 1.0000x reference)
import jax
import jax.numpy as jnp
from jax.experimental import pallas as pl


def kernel(x, edge_index, batch, ln_gamma, ln_beta, W1, b1, W2, b2, Wfc, bfc):
    raise NotImplementedError("write your pallas kernel here")


# speedup vs baseline: 11.1386x; 11.1386x over previous
import functools

import jax
import jax.numpy as jnp
from jax import lax
from jax.experimental import pallas as pl
from jax.experimental.pallas import tpu as pltpu
from jax.experimental.pallas import tpu_sc as plsc

N = 10000
E = 320000
G = 64
IN = 128
H = 16
OUT = 128

NCORES = 2
NSUB = 16
NW = NCORES * NSUB
CHUNK = 128
NCHUNK = -(-E // (NW * CHUNK))
EPAD = NW * CHUNK * NCHUNK
ROWS_PER_TILE = 632
NPAD = NSUB * ROWS_PER_TILE

_mesh = plsc.VectorSubcoreMesh(core_axis_name="c", subcore_axis_name="s")
_f32 = jnp.float32


def _sc_scatter_ones(dst_hbm, ones_hbm, zeros_hbm, out_hbm, dst_v, ones_v, acc):
    c = lax.axis_index("c").astype(jnp.int32)
    s = lax.axis_index("s").astype(jnp.int32)
    wid = s * jnp.int32(NCORES) + c
    stripe = pl.ds(s * jnp.int32(ROWS_PER_TILE), ROWS_PER_TILE)
    pltpu.sync_copy(zeros_hbm.at[stripe], acc.at[stripe])
    pltpu.sync_copy(dst_hbm.at[wid], dst_v)
    pltpu.sync_copy(ones_hbm, ones_v)
    plsc.subcore_barrier()

    @pl.loop(jnp.int32(0), jnp.int32(NCHUNK))
    def _(j):
        pltpu.sync_copy(ones_v, acc.at[dst_v.at[j]], add=True)

    plsc.subcore_barrier()
    pltpu.sync_copy(acc.at[stripe], out_hbm.at[c].at[stripe])


def _sc_gather_scatter(xs_hbm, src_hbm, dst_hbm, zeros_hbm, out_hbm,
                       src_v, dst_v, rows_v, acc):
    c = lax.axis_index("c").astype(jnp.int32)
    s = lax.axis_index("s").astype(jnp.int32)
    wid = s * jnp.int32(NCORES) + c
    stripe = pl.ds(s * jnp.int32(ROWS_PER_TILE), ROWS_PER_TILE)
    pltpu.sync_copy(zeros_hbm.at[stripe], acc.at[stripe])
    pltpu.sync_copy(src_hbm.at[wid], src_v)
    pltpu.sync_copy(dst_hbm.at[wid], dst_v)
    plsc.subcore_barrier()

    @pl.loop(jnp.int32(0), jnp.int32(NCHUNK))
    def _(j):
        pltpu.sync_copy(xs_hbm.at[src_v.at[j]], rows_v)
        pltpu.sync_copy(rows_v, acc.at[dst_v.at[j]], add=True)

    plsc.subcore_barrier()
    pltpu.sync_copy(acc.at[stripe], out_hbm.at[c].at[stripe])


LW = 128

_deg_kernel = pl.kernel(
    _sc_scatter_ones,
    out_type=jax.ShapeDtypeStruct((NCORES, NPAD, LW), _f32),
    mesh=_mesh,
    scratch_types=[
        pltpu.VMEM((NCHUNK, CHUNK), jnp.int32),
        pltpu.VMEM((CHUNK, LW), _f32),
        pltpu.VMEM_SHARED((NPAD, LW), _f32),
    ],
)

_agg_kernel = pl.kernel(
    _sc_gather_scatter,
    out_type=jax.ShapeDtypeStruct((NCORES, NPAD, LW), _f32),
    mesh=_mesh,
    scratch_types=[
        pltpu.VMEM((NCHUNK, CHUNK), jnp.int32),
        pltpu.VMEM((NCHUNK, CHUNK), jnp.int32),
        pltpu.VMEM((CHUNK, LW), _f32),
        pltpu.VMEM_SHARED((NPAD, LW), _f32),
    ],
)


def _tc_ln_matmul(x_ref, g_ref, b_ref, w1_ref, o_ref):
    x = x_ref[...]
    mu = jnp.mean(x, axis=1, keepdims=True)
    xc = x - mu
    var = jnp.mean(xc * xc, axis=1, keepdims=True)
    hn = xc * lax.rsqrt(var + 1e-5) * g_ref[...] + b_ref[...]
    o_ref[...] = jnp.dot(hn, w1_ref[...], preferred_element_type=_f32,
                         precision=lax.Precision.HIGHEST)


def _tc_scale(p0_ref, p1_ref, xw_ref, dinv_ref, xs_ref):
    deg = p0_ref[...] + p1_ref[...] + 1.0
    dinv = lax.rsqrt(deg)
    dinv_ref[...] = dinv
    xs_ref[...] = dinv * xw_ref[...]


def _tc_layer2(q0_ref, q1_ref, xs1_ref, dinv_ref, b1_ref, w2_ref, xs2_ref):
    dinv = dinv_ref[...]
    h1 = jnp.maximum(dinv * (q0_ref[...] + q1_ref[...] + xs1_ref[...])
                     + b1_ref[...], 0.0)
    xw2 = jnp.dot(h1, w2_ref[...], preferred_element_type=_f32,
                  precision=lax.Precision.HIGHEST)
    xs2_ref[...] = dinv * xw2


def _tc_head(r0_ref, r1_ref, xs2_ref, dinv_ref, b2_ref, batch_ref,
             wfc_ref, bfc_ref, o_ref):
    dinv = dinv_ref[...]
    h2 = jnp.maximum(dinv * (r0_ref[...] + r1_ref[...] + xs2_ref[...])
                     + b2_ref[...], 0.0)
    gids = lax.broadcasted_iota(jnp.int32, (G, N), 0)
    onehot = (gids == batch_ref[...]).astype(_f32)
    sums = jnp.dot(onehot, h2, preferred_element_type=_f32,
                   precision=lax.Precision.HIGHEST)
    cnts = jnp.sum(onehot, axis=1, keepdims=True)
    pooled = sums / jnp.maximum(cnts, 1.0)
    o_ref[...] = jnp.dot(pooled, wfc_ref[...], preferred_element_type=_f32,
                         precision=lax.Precision.HIGHEST) + bfc_ref[...]


def kernel(x, edge_index, batch, ln_gamma, ln_beta, W1, b1, W2, b2, Wfc, bfc):
    x = x.astype(_f32)
    src = edge_index[0].astype(jnp.int32)
    dst = edge_index[1].astype(jnp.int32)
    batch2d = batch.astype(jnp.int32).reshape(1, N)

    pad = EPAD - E
    srcp = jnp.concatenate([src, jnp.full((pad,), N, jnp.int32)])
    dstp = jnp.concatenate([dst, jnp.full((pad,), N, jnp.int32)])
    srcp = srcp.reshape(NW, NCHUNK, CHUNK)
    dstp = dstp.reshape(NW, NCHUNK, CHUNK)

    zeros_wide = jnp.zeros((NPAD, LW), _f32)
    ones_rows = jnp.ones((CHUNK, LW), _f32)

    degp = _deg_kernel(dstp, ones_rows, zeros_wide)

    xw1 = pl.pallas_call(
        _tc_ln_matmul,
        out_shape=jax.ShapeDtypeStruct((N, H), _f32),
    )(x, ln_gamma, ln_beta, W1)

    dinv16, xs1 = pl.pallas_call(
        _tc_scale,
        out_shape=(jax.ShapeDtypeStruct((N, H), _f32),
                   jax.ShapeDtypeStruct((N, H), _f32)),
    )(degp[0, :N, :H], degp[1, :N, :H], xw1)

    xs1_pad = jnp.pad(xs1, ((0, NPAD - N), (0, LW - H)))

    q = _agg_kernel(xs1_pad, srcp, dstp, zeros_wide)

    xs2 = pl.pallas_call(
        _tc_layer2,
        out_shape=jax.ShapeDtypeStruct((N, H), _f32),
    )(q[0, :N, :H], q[1, :N, :H], xs1, dinv16, b1, W2)

    xs2_pad = jnp.pad(xs2, ((0, NPAD - N), (0, LW - H)))

    r = _agg_kernel(xs2_pad, srcp, dstp, zeros_wide)

    out = pl.pallas_call(
        _tc_head,
        out_shape=jax.ShapeDtypeStruct((G, OUT), _f32),
    )(r[0, :N, :H], r[1, :N, :H], xs2, dinv16, b2, batch2d, Wfc, bfc)
    return out
